# baseline (speedup 1.0000x reference)
import functools

import jax
import jax.numpy as jnp
from jax.experimental import pallas as pl
from jax.experimental.pallas import tpu as pltpu

H = 4
C = 64
HC = H * C
NEG = -1e30


def _gelu(x):
    return 0.5 * x * (1.0 + jax.lax.erf(x * 0.7071067811865476))


def _lin_kernel(x_ref, w_ref, b_ref, o_ref, *, act):
    y = jnp.dot(x_ref[...], w_ref[...], preferred_element_type=jnp.float32)
    y = y + b_ref[...]
    if act:
        y = _gelu(y)
    o_ref[...] = y


def _linear(x, w, b, act=False, block_rows=None):
    n, fin = x.shape
    fout = w.shape[1]
    b2 = b.reshape(1, fout)
    kern = functools.partial(_lin_kernel, act=act)
    if block_rows is None:
        return pl.pallas_call(
            kern,
            out_shape=jax.ShapeDtypeStruct((n, fout), jnp.float32),
        )(x, w, b2)
    grid = (n // block_rows,)
    return pl.pallas_call(
        kern,
        grid=grid,
        in_specs=[
            pl.BlockSpec((block_rows, fin), lambda i: (i, 0)),
            pl.BlockSpec((fin, fout), lambda i: (0, 0)),
            pl.BlockSpec((1, fout), lambda i: (0, 0)),
        ],
        out_specs=pl.BlockSpec((block_rows, fout), lambda i: (i, 0)),
        out_shape=jax.ShapeDtypeStruct((n, fout), jnp.float32),
    )(x, w, b2)


def _lin2_kernel(x_ref, y_ref, w_ref, b_ref, o_ref):
    s = x_ref[...] + y_ref[...]
    o_ref[...] = jnp.dot(s, w_ref[...], preferred_element_type=jnp.float32) + b_ref[...]


def _linear_sum(x, y, w, b):
    n = x.shape[0]
    fout = w.shape[1]
    return pl.pallas_call(
        _lin2_kernel,
        out_shape=jax.ShapeDtypeStruct((n, fout), jnp.float32),
    )(x, y, w, b.reshape(1, fout))


def _bn_kernel(x_ref, g_ref, b_ref, o_ref, *, act):
    x = x_ref[...]
    mu = jnp.mean(x, axis=0, keepdims=True)
    var = jnp.mean((x - mu) ** 2, axis=0, keepdims=True)
    y = (x - mu) / jnp.sqrt(var + 1e-5) * g_ref[...] + b_ref[...]
    if act:
        y = _gelu(y)
    o_ref[...] = y


def _bn_act(x, g, b):
    n, f = x.shape
    return pl.pallas_call(
        functools.partial(_bn_kernel, act=True),
        out_shape=jax.ShapeDtypeStruct((n, f), jnp.float32),
    )(x, g.reshape(1, f), b.reshape(1, f))


def _edge_a_kernel(src_ref, dst_ref, ep_ref, xq_ref, xk_ref,
                   alpha_ref, m_ref, *, eb, n, etot):
    @pl.when(pl.program_id(0) == 0)
    def _():
        m_ref[...] = jnp.full((n, HC), NEG, jnp.float32)

    valid = jnp.clip(etot - pl.program_id(0) * eb, 0, eb)

    def body(j, carry):
        s = src_ref[j]
        d = dst_ref[j]
        q = xq_ref[pl.ds(d, 1), :]
        k = xk_ref[pl.ds(s, 1), :]
        e = ep_ref[pl.ds(j, 1), :]
        prod = q * (k + e)
        parts = []
        for h in range(H):
            sh = jnp.sum(prod[:, h * C:(h + 1) * C], axis=1, keepdims=True) * 0.125
            parts.append(jnp.broadcast_to(sh, (1, C)))
        ab = jnp.concatenate(parts, axis=1)
        alpha_ref[pl.ds(j, 1), :] = ab
        mrow = m_ref[pl.ds(d, 1), :]
        m_ref[pl.ds(d, 1), :] = jnp.maximum(mrow, ab)
        return carry

    jax.lax.fori_loop(0, valid, body, 0)


def _edge_b_kernel(src_ref, dst_ref, alpha_ref, ep_ref, xv_ref, m_ref,
                   num_ref, den_ref, *, eb, n, etot):
    @pl.when(pl.program_id(0) == 0)
    def _():
        num_ref[...] = jnp.zeros((n, HC), jnp.float32)
        den_ref[...] = jnp.zeros((n, HC), jnp.float32)

    valid = jnp.clip(etot - pl.program_id(0) * eb, 0, eb)

    def body(j, carry):
        s = src_ref[j]
        d = dst_ref[j]
        w = jnp.exp(alpha_ref[pl.ds(j, 1), :] - m_ref[pl.ds(d, 1), :])
        msg = w * (xv_ref[pl.ds(s, 1), :] + ep_ref[pl.ds(j, 1), :])
        num_ref[pl.ds(d, 1), :] = num_ref[pl.ds(d, 1), :] + msg
        den_ref[pl.ds(d, 1), :] = den_ref[pl.ds(d, 1), :] + w
        return carry

    jax.lax.fori_loop(0, valid, body, 0)


def _combine_kernel(num_ref, den_ref, xs_ref, o_ref):
    o_ref[...] = num_ref[...] / (den_ref[...] + 1e-16) + xs_ref[...]


def _agg_kernel(src_ref, dst_ref, h_ref, agg_ref, *, eb, n, etot):
    @pl.when(pl.program_id(0) == 0)
    def _():
        agg_ref[...] = jnp.zeros((n, HC), jnp.float32)

    valid = jnp.clip(etot - pl.program_id(0) * eb, 0, eb)

    def body(j, carry):
        s = src_ref[j]
        d = dst_ref[j]
        agg_ref[pl.ds(d, 1), :] = agg_ref[pl.ds(d, 1), :] + h_ref[pl.ds(s, 1), :]
        return carry

    jax.lax.fori_loop(0, valid, body, 0)


EB = 2048


def _tconv(x, src_p, dst_p, ea_p, etot, p):
    n = x.shape[0]
    epad = src_p.shape[0]
    e_blocks = epad // EB
    xq = _linear(x, p['Wq'], p['bq'])
    xk = _linear(x, p['Wk'], p['bk'])
    xv = _linear(x, p['Wv'], p['bv'])
    xs = _linear(x, p['Ws'], p['bs'])
    ep = _linear(ea_p, p['We'], jnp.zeros((HC,), jnp.float32), block_rows=EB)

    smem_spec = pl.BlockSpec((EB,), lambda i: (i,), memory_space=pltpu.SMEM)
    full = pl.BlockSpec((n, HC), lambda i: (0, 0))
    chunk = pl.BlockSpec((EB, HC), lambda i: (i, 0))

    alpha, m = pl.pallas_call(
        functools.partial(_edge_a_kernel, eb=EB, n=n, etot=etot),
        grid=(e_blocks,),
        in_specs=[smem_spec, smem_spec, chunk, full, full],
        out_specs=[chunk, full],
        out_shape=[jax.ShapeDtypeStruct((epad, HC), jnp.float32),
                   jax.ShapeDtypeStruct((n, HC), jnp.float32)],
    )(src_p, dst_p, ep, xq, xk)

    num, den = pl.pallas_call(
        functools.partial(_edge_b_kernel, eb=EB, n=n, etot=etot),
        grid=(e_blocks,),
        in_specs=[smem_spec, smem_spec, chunk, chunk, full, full],
        out_specs=[full, full],
        out_shape=[jax.ShapeDtypeStruct((n, HC), jnp.float32),
                   jax.ShapeDtypeStruct((n, HC), jnp.float32)],
    )(src_p, dst_p, alpha, ep, xv, m)

    return pl.pallas_call(
        _combine_kernel,
        out_shape=jax.ShapeDtypeStruct((n, HC), jnp.float32),
    )(num, den, xs)


def _gin_agg(h, src_p, dst_p, etot):
    n = h.shape[0]
    e_blocks = src_p.shape[0] // EB
    smem_spec = pl.BlockSpec((EB,), lambda i: (i,), memory_space=pltpu.SMEM)
    full = pl.BlockSpec((n, HC), lambda i: (0, 0))
    return pl.pallas_call(
        functools.partial(_agg_kernel, eb=EB, n=n, etot=etot),
        grid=(e_blocks,),
        in_specs=[smem_spec, smem_spec, full],
        out_specs=full,
        out_shape=jax.ShapeDtypeStruct((n, HC), jnp.float32),
    )(src_p, dst_p, h)


def _pool_kernel(batch_ref, h_ref, pmax_ref, pmean_ref, psum, pcnt, *, n, b):
    f = h_ref.shape[1]
    pmax_ref[...] = jnp.full((b, f), NEG, jnp.float32)
    psum[...] = jnp.zeros((b, f), jnp.float32)
    pcnt[...] = jnp.zeros((b, f), jnp.float32)
    ones = jnp.ones((1, f), jnp.float32)

    def body(j, carry):
        g = batch_ref[j]
        row = h_ref[pl.ds(j, 1), :]
        pmax_ref[pl.ds(g, 1), :] = jnp.maximum(pmax_ref[pl.ds(g, 1), :], row)
        psum[pl.ds(g, 1), :] = psum[pl.ds(g, 1), :] + row
        pcnt[pl.ds(g, 1), :] = pcnt[pl.ds(g, 1), :] + ones
        return carry

    jax.lax.fori_loop(0, n, body, 0)
    mx = pmax_ref[...]
    pmax_ref[...] = jnp.where(mx > (0.5 * NEG), mx, 0.0)
    pmean_ref[...] = psum[...] / jnp.maximum(pcnt[...], 1.0)


def _pool(h, batch, b):
    n, f = h.shape
    return pl.pallas_call(
        functools.partial(_pool_kernel, n=n, b=b),
        in_specs=[pl.BlockSpec(memory_space=pltpu.SMEM),
                  pl.BlockSpec((n, f), lambda: (0, 0))],
        out_specs=[pl.BlockSpec((b, f), lambda: (0, 0)),
                   pl.BlockSpec((b, f), lambda: (0, 0))],
        out_shape=[jax.ShapeDtypeStruct((b, f), jnp.float32),
                   jax.ShapeDtypeStruct((b, f), jnp.float32)],
        scratch_shapes=[pltpu.VMEM((b, f), jnp.float32),
                        pltpu.VMEM((b, f), jnp.float32)],
    )(batch, h)


def _head_kernel(h_ref, w1, c1, w2, c2, w3, c3, wo, co, o_ref):
    z = _gelu(jnp.dot(h_ref[...], w1[...], preferred_element_type=jnp.float32) + c1[...])
    z = _gelu(jnp.dot(z, w2[...], preferred_element_type=jnp.float32) + c2[...])
    z = _gelu(jnp.dot(z, w3[...], preferred_element_type=jnp.float32) + c3[...])
    o_ref[...] = jnp.dot(z, wo[...], preferred_element_type=jnp.float32) + co[...]


def _head(hybrid, p):
    b = hybrid.shape[0]
    return pl.pallas_call(
        _head_kernel,
        out_shape=jax.ShapeDtypeStruct((b, 1), jnp.float32),
    )(hybrid, p['W1'], p['c1'].reshape(1, -1), p['W2'], p['c2'].reshape(1, -1),
      p['W3'], p['c3'].reshape(1, -1), p['Wo'], p['co'].reshape(1, -1))


def _block(x, ei, ea, batch, p, b):
    src = ei[0].astype(jnp.int32)
    dst = ei[1].astype(jnp.int32)
    etot = src.shape[0]
    epad = ((etot + EB - 1) // EB) * EB
    pad = epad - etot
    src = jnp.pad(src, (0, pad))
    dst = jnp.pad(dst, (0, pad))
    ea = jnp.pad(ea, ((0, pad), (0, 0)))
    h = _tconv(x, src, dst, ea, etot, p['t1'])
    h = _bn_act(h, p['g1'], p['b1'])
    h = _tconv(h, src, dst, ea, etot, p['t2'])
    h = _bn_act(h, p['g2'], p['b2'])
    agg = _gin_agg(h, src, dst, etot)
    h = _linear_sum(h, agg, p['Wg'], p['bg'])
    h = _bn_act(h, p['g3'], p['b3'])
    pmax, pmean = _pool(h, batch.astype(jnp.int32), b)
    return jnp.concatenate([pmax, pmean], axis=1)


def kernel(x1, edge_index1, batch_index1, descriptors1, edge_attr1,
           x2, edge_index2, batch_index2, descriptors2, edge_attr2, params):
    b = descriptors1.shape[0]
    h1 = _block(x1, edge_index1, edge_attr1, batch_index1, params, b)
    h2 = _block(x2, edge_index2, edge_attr2, batch_index2, params, b)
    hybrid = jnp.concatenate([h1, h2, descriptors1, descriptors2], axis=1)
    out = _head(hybrid, params)
    return out, hybrid
